# Initial kernel scaffold; baseline (speedup 1.0000x reference)
import jax
import jax.numpy as jnp
from jax.experimental import pallas as pl


def kernel(loss_field1, loss_field2, loss_presence, loss_presence_neg, matching, seq_lens, presence_logits, matching_priors, matching_thresholds):
    raise NotImplementedError("write your pallas kernel here")


# speedup vs baseline: 1.6848x; 1.6848x over previous
import functools

import jax
import jax.numpy as jnp
from jax import lax
from jax.experimental import pallas as pl
from jax.experimental.pallas import tpu as pltpu

_MOM = 0.1
_B, _L, _K, _C = 8, 2048, 8, 16
_N = _B * _L
_ROWS = 512
_GRID = _N // _ROWS
_BPG = _L // _ROWS
_IMAX = 2147483647


def _phase1_body(seq_ref, l1_ref, l2_ref, lp_ref, ln_ref, mt_ref, pri_ref,
                 f1_ref, f2_ref, po_ref, pro_ref, acc_ref):
    g = pl.program_id(0)

    @pl.when(g == 0)
    def _init():
        acc_ref[...] = jnp.zeros_like(acc_ref)

    m = mt_ref[...]
    maskb = (m >= 0).astype(jnp.int32)
    mclip = jnp.maximum(m, 0)
    ksh = lax.broadcasted_iota(jnp.int32, (_ROWS, _K), 1)
    packed_m = jnp.sum(mclip << (ksh * 4), axis=1, keepdims=True)
    packed_k = jnp.sum(maskb << ksh, axis=1, keepdims=True)

    jl = lax.broadcasted_iota(jnp.int32, (_ROWS, _K * _C), 1)
    kid = jl >> 4
    cid = jl & 15
    me = (packed_m >> (kid * 4)) & 15
    ke = ((packed_k >> kid) & 1).astype(jnp.float32)
    oh = (me == cid).astype(jnp.float32)
    w = oh * ke

    seq_b = seq_ref[g // _BPG]
    l_loc = (g % _BPG) * _ROWS + lax.broadcasted_iota(
        jnp.int32, (_ROWS, _K * _C), 0)
    idxf = (l_loc < seq_b).astype(jnp.float32)

    lane0 = (jl == 0).astype(jnp.float32)

    acc_ref[0:1, :] += jnp.sum(l1_ref[...] * w, axis=0, keepdims=True)
    acc_ref[1:2, :] += jnp.sum(l2_ref[...] * w, axis=0, keepdims=True)
    pres = (lp_ref[...] * ke - ln_ref[...] * (1.0 - ke)) * oh * idxf
    acc_ref[2:3, :] += jnp.sum(pres, axis=0, keepdims=True)
    acc_ref[3:4, :] += jnp.sum(w, axis=0, keepdims=True)
    acc_ref[4:5, :] += jnp.sum(w * idxf, axis=0, keepdims=True)
    acc_ref[5:6, :] += jnp.sum(idxf * lane0, axis=0, keepdims=True)

    @pl.when(g == _GRID - 1)
    def _fin():
        a = acc_ref[...]
        s1 = jnp.sum(a[0:1, :])
        s2 = jnp.sum(a[1:2, :])
        sp = jnp.sum(a[2:3, :])
        mc = jnp.sum(a[3:4, :])
        ic = jnp.sum(a[5:6, :])
        mcount = jnp.maximum(mc, 1.0)
        icount = jnp.maximum(ic * _K, 1.0)
        f1_ref[...] = jnp.full((1, 1), s1 / mcount, jnp.float32)
        f2_ref[...] = jnp.full((1, 1), s2 / mcount, jnp.float32)
        po_ref[...] = jnp.full((1, 1), sp / icount, jnp.float32)
        krow = a[4:5, :]
        kid_r = lax.broadcasted_iota(jnp.int32, (1, _K * _C), 1) >> 4
        lane8 = lax.broadcasted_iota(jnp.int32, (1, _K), 1)
        means = jnp.zeros((1, _K), jnp.float32)
        for k in range(_K):
            mk = jnp.sum(krow * (kid_r == k).astype(jnp.float32)) / ic
            means = means + mk * (lane8 == k).astype(jnp.float32)
        pro_ref[...] = pri_ref[...] * (1.0 - _MOM) + means * _MOM


def _phase2_body(plT_ref, seq_ref, pri_ref, thr_ref, out_ref, keys_ref):
    x = plT_ref[...]
    bits = lax.bitcast_convert_type(x, jnp.int32)
    keys = jnp.where(bits < 0, bits ^ jnp.int32(0x7FFFFFFF), bits)

    nlane = lax.broadcasted_iota(jnp.int32, (_K, _N), 1)
    ll = nlane & (_L - 1)
    bb = nlane >> 11
    valid = jnp.zeros((_K, _N), jnp.bool_)
    cnt_total = jnp.int32(0)
    for b in range(_B):
        sb = jnp.minimum(seq_ref[b], _L)
        valid = jnp.logical_or(valid, jnp.logical_and(bb == b, ll < sb))
        cnt_total = cnt_total + sb
    keys = jnp.where(valid, keys, _IMAX)

    nf = cnt_total.astype(jnp.float32)
    ind = (1.0 - pri_ref[...]) * nf
    nm1 = cnt_total - 1
    rb = jnp.clip(jnp.floor(ind).astype(jnp.int32), 0, nm1)
    ru = jnp.clip(jnp.ceil(ind).astype(jnp.int32), 0, nm1)

    keys_ref[...] = keys

    def _step(_, carry):
        lo, hi = carry
        mid = (lo >> 1) + (hi >> 1) + (lo & hi & 1)
        cnt = jnp.sum((keys_ref[...] <= mid).astype(jnp.int32),
                      axis=1, keepdims=True)
        pred = cnt >= rb + 1
        return jnp.where(pred, lo, mid + 1), jnp.where(pred, mid, hi)

    lo0 = jnp.full((_K, 1), jnp.int32(-2147483647) - 1)
    hi0 = jnp.full((_K, 1), _IMAX, jnp.int32)
    keyb, _ = lax.fori_loop(0, 32, _step, (lo0, hi0))

    kk = keys_ref[...]
    cnt_b = jnp.sum((kk <= keyb).astype(jnp.int32), axis=1, keepdims=True)
    above = jnp.min(jnp.where(kk > keyb, kk, _IMAX), axis=1, keepdims=True)
    keyu = jnp.where(cnt_b >= ru + 1, keyb, above)

    def _unkey(kv):
        return lax.bitcast_convert_type(
            jnp.where(kv < 0, kv ^ jnp.int32(0x7FFFFFFF), kv), jnp.float32)

    q = 0.5 * (_unkey(keyb) + _unkey(keyu))
    out_ref[...] = thr_ref[...] * (1.0 - _MOM) + q * _MOM


def kernel(loss_field1, loss_field2, loss_presence, loss_presence_neg,
           matching, seq_lens, presence_logits,
           matching_priors, matching_thresholds):
    l1 = loss_field1.reshape(_N, _K * _C)
    l2 = loss_field2.reshape(_N, _K * _C)
    lp = loss_presence.reshape(_N, _K * _C)
    ln = loss_presence_neg.reshape(_N, _K * _C)
    mt = matching.reshape(_N, _K)
    pri = matching_priors.reshape(1, _K)

    row_spec = pl.BlockSpec((_ROWS, _K * _C), lambda g: (g, 0))
    out11 = pl.BlockSpec((1, 1), lambda g: (0, 0))
    f1, f2, po, pro = pl.pallas_call(
        _phase1_body,
        grid=(_GRID,),
        in_specs=[
            pl.BlockSpec(memory_space=pltpu.SMEM),
            row_spec, row_spec, row_spec, row_spec,
            pl.BlockSpec((_ROWS, _K), lambda g: (g, 0)),
            pl.BlockSpec((1, _K), lambda g: (0, 0)),
        ],
        out_specs=[out11, out11, out11, pl.BlockSpec((1, _K), lambda g: (0, 0))],
        out_shape=[
            jax.ShapeDtypeStruct((1, 1), jnp.float32),
            jax.ShapeDtypeStruct((1, 1), jnp.float32),
            jax.ShapeDtypeStruct((1, 1), jnp.float32),
            jax.ShapeDtypeStruct((1, _K), jnp.float32),
        ],
        scratch_shapes=[pltpu.VMEM((8, _K * _C), jnp.float32)],
    )(seq_lens, l1, l2, lp, ln, mt, pri)

    plT = presence_logits.reshape(_N, _K).T
    thr = pl.pallas_call(
        _phase2_body,
        in_specs=[
            pl.BlockSpec((_K, _N), lambda: (0, 0)),
            pl.BlockSpec(memory_space=pltpu.SMEM),
            pl.BlockSpec((_K, 1), lambda: (0, 0)),
            pl.BlockSpec((_K, 1), lambda: (0, 0)),
        ],
        out_specs=pl.BlockSpec((_K, 1), lambda: (0, 0)),
        out_shape=jax.ShapeDtypeStruct((_K, 1), jnp.float32),
        scratch_shapes=[pltpu.VMEM((_K, _N), jnp.int32)],
    )(plT, seq_lens, matching_priors.reshape(_K, 1),
      matching_thresholds.reshape(_K, 1))

    return (f1[0, 0], f2[0, 0], po[0, 0], pro[0], thr[:, 0])


# speedup vs baseline: 3.0097x; 1.7864x over previous
import functools

import jax
import jax.numpy as jnp
from jax import lax
from jax.experimental import pallas as pl
from jax.experimental.pallas import tpu as pltpu

_MOM = 0.1
_B, _L, _K, _C = 8, 2048, 8, 16
_N = _B * _L
_ROWS = 512
_GRID = _N // _ROWS
_BPG = _L // _ROWS
_IMAX = 2147483647


def _phase1_body(seq_ref, l1_ref, l2_ref, lp_ref, ln_ref, mt_ref, pri_ref,
                 f1_ref, f2_ref, po_ref, pro_ref, acc_ref):
    g = pl.program_id(0)

    @pl.when(g == 0)
    def _init():
        acc_ref[...] = jnp.zeros_like(acc_ref)

    m = mt_ref[0]
    maskb = (m >= 0).astype(jnp.int32)
    mclip = jnp.maximum(m, 0)
    ksh = lax.broadcasted_iota(jnp.int32, (_ROWS, _K), 1)
    packed_m = jnp.sum(mclip << (ksh * 4), axis=1, keepdims=True)
    packed_k = jnp.sum(maskb << ksh, axis=1, keepdims=True)

    jl = lax.broadcasted_iota(jnp.int32, (_ROWS, _K * _C), 1)
    kid = jl >> 4
    cid = jl & 15
    me = (packed_m >> (kid * 4)) & 15
    ke = ((packed_k >> kid) & 1).astype(jnp.float32)
    oh = (me == cid).astype(jnp.float32)
    w = oh * ke

    seq_b = seq_ref[g // _BPG]
    l_loc = (g % _BPG) * _ROWS + lax.broadcasted_iota(
        jnp.int32, (_ROWS, _K * _C), 0)
    idxf = (l_loc < seq_b).astype(jnp.float32)

    lane0 = (jl == 0).astype(jnp.float32)

    acc_ref[0:1, :] += jnp.sum(l1_ref[0] * w, axis=0, keepdims=True)
    acc_ref[1:2, :] += jnp.sum(l2_ref[0] * w, axis=0, keepdims=True)
    pres = (lp_ref[0] * ke - ln_ref[0] * (1.0 - ke)) * oh * idxf
    acc_ref[2:3, :] += jnp.sum(pres, axis=0, keepdims=True)
    acc_ref[3:4, :] += jnp.sum(w, axis=0, keepdims=True)
    acc_ref[4:5, :] += jnp.sum(w * idxf, axis=0, keepdims=True)
    acc_ref[5:6, :] += jnp.sum(idxf * lane0, axis=0, keepdims=True)

    @pl.when(g == _GRID - 1)
    def _fin():
        a = acc_ref[...]
        s1 = jnp.sum(a[0:1, :])
        s2 = jnp.sum(a[1:2, :])
        sp = jnp.sum(a[2:3, :])
        mc = jnp.sum(a[3:4, :])
        ic = jnp.sum(a[5:6, :])
        mcount = jnp.maximum(mc, 1.0)
        icount = jnp.maximum(ic * _K, 1.0)
        f1_ref[...] = jnp.full((1, 1), s1 / mcount, jnp.float32)
        f2_ref[...] = jnp.full((1, 1), s2 / mcount, jnp.float32)
        po_ref[...] = jnp.full((1, 1), sp / icount, jnp.float32)
        krow = a[4:5, :]
        kid_r = lax.broadcasted_iota(jnp.int32, (1, _K * _C), 1) >> 4
        lane8 = lax.broadcasted_iota(jnp.int32, (1, _K), 1)
        means = jnp.zeros((1, _K), jnp.float32)
        for k in range(_K):
            mk = jnp.sum(krow * (kid_r == k).astype(jnp.float32)) / ic
            means = means + mk * (lane8 == k).astype(jnp.float32)
        pro_ref[...] = pri_ref[...] * (1.0 - _MOM) + means * _MOM


def _phase2_body(plT_ref, seq_ref, pri_ref, thr_ref, out_ref, keys_ref):
    x = plT_ref[...]
    bits = lax.bitcast_convert_type(x, jnp.int32)
    keys = jnp.where(bits < 0, bits ^ jnp.int32(0x7FFFFFFF), bits)

    nlane = lax.broadcasted_iota(jnp.int32, (_K, _N), 1)
    ll = nlane & (_L - 1)
    bb = nlane >> 11
    valid = jnp.zeros((_K, _N), jnp.bool_)
    cnt_total = jnp.int32(0)
    for b in range(_B):
        sb = jnp.minimum(seq_ref[b], _L)
        valid = jnp.logical_or(valid, jnp.logical_and(bb == b, ll < sb))
        cnt_total = cnt_total + sb
    keys = jnp.where(valid, keys, _IMAX)

    nf = cnt_total.astype(jnp.float32)
    ind = (1.0 - pri_ref[...]) * nf
    nm1 = cnt_total - 1
    rb = jnp.clip(jnp.floor(ind).astype(jnp.int32), 0, nm1)
    ru = jnp.clip(jnp.ceil(ind).astype(jnp.int32), 0, nm1)

    keys_ref[...] = keys

    def _step(_, carry):
        lo, hi = carry
        mid = (lo >> 1) + (hi >> 1) + (lo & hi & 1)
        cnt = jnp.sum((keys_ref[...] <= mid).astype(jnp.int32),
                      axis=1, keepdims=True)
        pred = cnt >= rb + 1
        return jnp.where(pred, lo, mid + 1), jnp.where(pred, mid, hi)

    lo0 = jnp.full((_K, 1), jnp.int32(-2147483647) - 1)
    hi0 = jnp.full((_K, 1), _IMAX, jnp.int32)
    keyb, _ = lax.fori_loop(0, 32, _step, (lo0, hi0))

    kk = keys_ref[...]
    cnt_b = jnp.sum((kk <= keyb).astype(jnp.int32), axis=1, keepdims=True)
    above = jnp.min(jnp.where(kk > keyb, kk, _IMAX), axis=1, keepdims=True)
    keyu = jnp.where(cnt_b >= ru + 1, keyb, above)

    def _unkey(kv):
        return lax.bitcast_convert_type(
            jnp.where(kv < 0, kv ^ jnp.int32(0x7FFFFFFF), kv), jnp.float32)

    q = 0.5 * (_unkey(keyb) + _unkey(keyu))
    out_ref[...] = thr_ref[...] * (1.0 - _MOM) + q * _MOM


def kernel(loss_field1, loss_field2, loss_presence, loss_presence_neg,
           matching, seq_lens, presence_logits,
           matching_priors, matching_thresholds):
    l1 = loss_field1.reshape(_B, _L, _K * _C)
    l2 = loss_field2.reshape(_B, _L, _K * _C)
    lp = loss_presence.reshape(_B, _L, _K * _C)
    ln = loss_presence_neg.reshape(_B, _L, _K * _C)
    mt = matching
    pri = matching_priors.reshape(1, _K)

    row_spec = pl.BlockSpec((1, _ROWS, _K * _C),
                            lambda g: (g // _BPG, g % _BPG, 0))
    out11 = pl.BlockSpec((1, 1), lambda g: (0, 0))
    f1, f2, po, pro = pl.pallas_call(
        _phase1_body,
        grid=(_GRID,),
        in_specs=[
            pl.BlockSpec(memory_space=pltpu.SMEM),
            row_spec, row_spec, row_spec, row_spec,
            pl.BlockSpec((1, _ROWS, _K), lambda g: (g // _BPG, g % _BPG, 0)),
            pl.BlockSpec((1, _K), lambda g: (0, 0)),
        ],
        out_specs=[out11, out11, out11, pl.BlockSpec((1, _K), lambda g: (0, 0))],
        out_shape=[
            jax.ShapeDtypeStruct((1, 1), jnp.float32),
            jax.ShapeDtypeStruct((1, 1), jnp.float32),
            jax.ShapeDtypeStruct((1, 1), jnp.float32),
            jax.ShapeDtypeStruct((1, _K), jnp.float32),
        ],
        scratch_shapes=[pltpu.VMEM((8, _K * _C), jnp.float32)],
    )(seq_lens, l1, l2, lp, ln, mt, pri)

    plT = presence_logits.reshape(_N, _K).T
    thr = pl.pallas_call(
        _phase2_body,
        in_specs=[
            pl.BlockSpec((_K, _N), lambda: (0, 0)),
            pl.BlockSpec(memory_space=pltpu.SMEM),
            pl.BlockSpec((_K, 1), lambda: (0, 0)),
            pl.BlockSpec((_K, 1), lambda: (0, 0)),
        ],
        out_specs=pl.BlockSpec((_K, 1), lambda: (0, 0)),
        out_shape=jax.ShapeDtypeStruct((_K, 1), jnp.float32),
        scratch_shapes=[pltpu.VMEM((_K, _N), jnp.int32)],
    )(plT, seq_lens, matching_priors.reshape(_K, 1),
      matching_thresholds.reshape(_K, 1))

    return (f1[0, 0], f2[0, 0], po[0, 0], pro[0], thr[:, 0])
